# Initial kernel scaffold; baseline (speedup 1.0000x reference)
import jax
import jax.numpy as jnp
from jax.experimental import pallas as pl


def kernel(x, edge_index, W, b):
    raise NotImplementedError("write your pallas kernel here")


# speedup vs baseline: 15.2821x; 15.2821x over previous
import jax
import jax.numpy as jnp
from jax import lax
from jax.experimental import pallas as pl
from jax.experimental.pallas import tpu as pltpu
from jax.experimental.pallas import tpu_sc as plsc

N = 10000
E = 320000
CH = 128
HCH = CH // 2

NC = 2
NS = 16
NW = NC * NS

C = 128
KD = 80
KA = 160
NPAD = 10112
RPT = NPAD // NS
OLAST = N - (NS - 1) * RPT

_mesh = plsc.VectorSubcoreMesh(
    core_axis_name="c", subcore_axis_name="s", num_cores=NC, num_subcores=NS)


def _striped_out_copy(s, sh_ref, out_view):
  @pl.when(s < NS - 1)
  def _():
    pltpu.sync_copy(sh_ref.at[pl.ds(s * RPT, RPT)],
                    out_view.at[pl.ds(s * RPT, RPT)])

  @pl.when(s == NS - 1)
  def _():
    pltpu.sync_copy(sh_ref.at[pl.ds((NS - 1) * RPT, OLAST)],
                    out_view.at[pl.ds((NS - 1) * RPT, OLAST)])


def _deg_body(dst_hbm, zeros_hbm, out_hbm, dst_v, ones_v, deg_sh):
  c = lax.axis_index("c")
  s = lax.axis_index("s")
  wid = s * NC + c
  pltpu.sync_copy(zeros_hbm.at[pl.ds(s * RPT, RPT)],
                  deg_sh.at[pl.ds(s * RPT, RPT)])
  pltpu.sync_copy(dst_hbm.at[wid], dst_v)

  def fill(i, carry):
    ones_v[i, :] = jnp.ones((16,), jnp.float32)
    return carry

  lax.fori_loop(0, C, fill, 0)
  plsc.subcore_barrier()

  def body(j, carry):
    pltpu.sync_copy(ones_v, deg_sh.at[dst_v.at[j]], add=True)
    return carry

  lax.fori_loop(0, KD, body, 0)
  plsc.subcore_barrier()
  _striped_out_copy(s, deg_sh, out_hbm.at[c])


_sc_deg = pl.kernel(
    _deg_body,
    out_type=jax.ShapeDtypeStruct((NC, N, 16), jnp.float32),
    mesh=_mesh,
    scratch_types=[
        pltpu.VMEM((KD, C), jnp.int32),
        pltpu.VMEM((C, 16), jnp.float32),
        pltpu.VMEM_SHARED((NPAD, 16), jnp.float32),
    ],
)


def _agg_body(z2_hbm, src_hbm, dst_hbm, zeros_hbm, out_hbm,
              src_v, dst_v, buf0, buf1, acc, sem0, sem1):
  c = lax.axis_index("c")
  s = lax.axis_index("s")
  pltpu.sync_copy(zeros_hbm.at[pl.ds(s * RPT, RPT)],
                  acc.at[pl.ds(s * RPT, RPT)])
  pltpu.sync_copy(src_hbm.at[c, s], src_v)
  pltpu.sync_copy(dst_hbm.at[s], dst_v)
  plsc.subcore_barrier()

  pltpu.async_copy(z2_hbm.at[src_v.at[0]], buf0, sem0)
  pltpu.async_copy(z2_hbm.at[src_v.at[1]], buf1, sem1)

  def body(i, carry):
    j0 = 2 * i
    j1 = 2 * i + 1
    pltpu.make_async_copy(z2_hbm.at[src_v.at[j0]], buf0, sem0).wait()
    pltpu.sync_copy(buf0, acc.at[dst_v.at[j0]], add=True)

    @pl.when(i < KA // 2 - 1)
    def _():
      pltpu.async_copy(z2_hbm.at[src_v.at[j0 + 2]], buf0, sem0)

    pltpu.make_async_copy(z2_hbm.at[src_v.at[j1]], buf1, sem1).wait()
    pltpu.sync_copy(buf1, acc.at[dst_v.at[j1]], add=True)

    @pl.when(i < KA // 2 - 1)
    def _():
      pltpu.async_copy(z2_hbm.at[src_v.at[j1 + 2]], buf1, sem1)

    return carry

  lax.fori_loop(0, KA // 2, body, 0)
  plsc.subcore_barrier()
  _striped_out_copy(s, acc, out_hbm.at[c])


_sc_agg = pl.kernel(
    _agg_body,
    out_type=jax.ShapeDtypeStruct((NC, N, HCH), jnp.float32),
    mesh=_mesh,
    scratch_types=[
        pltpu.VMEM((KA, C), jnp.int32),
        pltpu.VMEM((KA, C), jnp.int32),
        pltpu.VMEM((C, HCH), jnp.float32),
        pltpu.VMEM((C, HCH), jnp.float32),
        pltpu.VMEM_SHARED((NPAD, HCH), jnp.float32),
        pltpu.SemaphoreType.DMA,
        pltpu.SemaphoreType.DMA,
    ],
    compiler_params=pltpu.CompilerParams(use_tc_tiling_on_sc=False),
)


BN = 1000


def _scale_body(x_ref, d_ref, z_ref):
  deg = 1.0 + d_ref[0, :, 0:1] + d_ref[1, :, 0:1]
  z_ref[...] = x_ref[...] * lax.rsqrt(deg)


_tc_scale = pl.pallas_call(
    _scale_body,
    grid=(N // BN,),
    in_specs=[
        pl.BlockSpec((BN, CH), lambda i: (i, 0)),
        pl.BlockSpec((NC, BN, 16), lambda i: (0, i, 0)),
    ],
    out_specs=pl.BlockSpec((BN, CH), lambda i: (i, 0)),
    out_shape=jax.ShapeDtypeStruct((N, CH), jnp.float32),
)


def _final_body(a_ref, z_ref, d_ref, w_ref, b_ref, o_ref):
  deg = 1.0 + d_ref[0, :, 0:1] + d_ref[1, :, 0:1]
  agg = jnp.concatenate([a_ref[0], a_ref[1]], axis=1)
  t = (agg + z_ref[...]) * lax.rsqrt(deg)
  o_ref[...] = jnp.dot(t, w_ref[...],
                       preferred_element_type=jnp.float32) + b_ref[...]


_tc_final = pl.pallas_call(
    _final_body,
    grid=(N // BN,),
    in_specs=[
        pl.BlockSpec((NC, BN, HCH), lambda i: (0, i, 0)),
        pl.BlockSpec((BN, CH), lambda i: (i, 0)),
        pl.BlockSpec((NC, BN, 16), lambda i: (0, i, 0)),
        pl.BlockSpec((CH, CH), lambda i: (0, 0)),
        pl.BlockSpec((1, CH), lambda i: (0, 0)),
    ],
    out_specs=pl.BlockSpec((BN, CH), lambda i: (i, 0)),
    out_shape=jax.ShapeDtypeStruct((N, CH), jnp.float32),
)


@jax.jit
def kernel(x, edge_index, W, b):
  src = edge_index[0].astype(jnp.int32)
  dst = edge_index[1].astype(jnp.int32)

  epd = E // NW
  dstd = jnp.concatenate(
      [dst.reshape(NW, epd),
       jnp.full((NW, KD * C - epd), N, jnp.int32)], axis=1).reshape(NW, KD, C)

  epa = E // NS
  pada = KA * C - epa
  s2 = 2 * src.reshape(NS, epa)
  srca = jnp.stack([
      jnp.concatenate([s2, jnp.zeros((NS, pada), jnp.int32)], axis=1),
      jnp.concatenate([s2 + 1, jnp.ones((NS, pada), jnp.int32)], axis=1),
  ]).reshape(NC, NS, KA, C)
  dsta = jnp.concatenate(
      [dst.reshape(NS, epa),
       jnp.full((NS, pada), N, jnp.int32)], axis=1).reshape(NS, KA, C)

  zeros16 = jnp.zeros((NPAD, 16), jnp.float32)
  zeros64 = jnp.zeros((NPAD, HCH), jnp.float32)

  degp = _sc_deg(dstd, zeros16)
  z = _tc_scale(x, degp)
  accp = _sc_agg(z.reshape(2 * N, HCH), srca, dsta, zeros64)
  out = _tc_final(accp, z, degp, W, b.reshape(1, CH))
  return out


# speedup vs baseline: 15.6784x; 1.0259x over previous
import jax
import jax.numpy as jnp
from jax import lax
from jax.experimental import pallas as pl
from jax.experimental.pallas import tpu as pltpu
from jax.experimental.pallas import tpu_sc as plsc

N = 10000
E = 320000
CH = 128
HCH = CH // 2

NC = 2
NS = 16
NW = NC * NS

C = 128
KD = 80
KA = 160
NPAD = 10112
RPT = NPAD // NS
OLAST = N - (NS - 1) * RPT

_mesh = plsc.VectorSubcoreMesh(
    core_axis_name="c", subcore_axis_name="s", num_cores=NC, num_subcores=NS)


def _striped_out_copy(s, sh_ref, out_view):
  @pl.when(s < NS - 1)
  def _():
    pltpu.sync_copy(sh_ref.at[pl.ds(s * RPT, RPT)],
                    out_view.at[pl.ds(s * RPT, RPT)])

  @pl.when(s == NS - 1)
  def _():
    pltpu.sync_copy(sh_ref.at[pl.ds((NS - 1) * RPT, OLAST)],
                    out_view.at[pl.ds((NS - 1) * RPT, OLAST)])


def _deg_body(dst_hbm, zeros_hbm, out_hbm, dst_v, ones_v, deg_sh):
  c = lax.axis_index("c")
  s = lax.axis_index("s")
  wid = s * NC + c
  pltpu.sync_copy(zeros_hbm.at[pl.ds(s * RPT, RPT)],
                  deg_sh.at[pl.ds(s * RPT, RPT)])
  pltpu.sync_copy(dst_hbm.at[wid], dst_v)

  def fill(i, carry):
    ones_v[i, :] = jnp.ones((16,), jnp.float32)
    return carry

  lax.fori_loop(0, C, fill, 0)
  plsc.subcore_barrier()

  def body(j, carry):
    pltpu.sync_copy(ones_v, deg_sh.at[dst_v.at[j]], add=True)
    return carry

  lax.fori_loop(0, KD, body, 0)
  plsc.subcore_barrier()
  _striped_out_copy(s, deg_sh, out_hbm.at[c])


_sc_deg = pl.kernel(
    _deg_body,
    out_type=jax.ShapeDtypeStruct((NC, N, 16), jnp.float32),
    mesh=_mesh,
    scratch_types=[
        pltpu.VMEM((KD, C), jnp.int32),
        pltpu.VMEM((C, 16), jnp.float32),
        pltpu.VMEM_SHARED((NPAD, 16), jnp.float32),
    ],
)


NB = 4


def _agg_body(z2_hbm, src_hbm, dst_hbm, zeros_hbm, out_hbm,
              src_v, dst_v, bufs, gsems, ssems, acc):
  c = lax.axis_index("c")
  s = lax.axis_index("s")
  pltpu.sync_copy(zeros_hbm.at[pl.ds(s * RPT, RPT)],
                  acc.at[pl.ds(s * RPT, RPT)])
  pltpu.sync_copy(src_hbm.at[c, s], src_v)
  pltpu.sync_copy(dst_hbm.at[s], dst_v)
  plsc.subcore_barrier()

  for b in range(NB):
    pltpu.async_copy(z2_hbm.at[src_v.at[b]], bufs[b], gsems[b])

  def body(g, carry):
    j = g * NB
    for b in range(NB):
      pltpu.make_async_copy(z2_hbm.at[src_v.at[j + b]], bufs[b],
                            gsems[b]).wait()
      pltpu.async_copy(bufs[b], acc.at[dst_v.at[j + b]], ssems[b], add=True)
    @pl.when(g < KA // NB - 1)
    def _():
      for b in range(NB):
        pltpu.make_async_copy(bufs[b], acc.at[dst_v.at[j + b]],
                              ssems[b]).wait()
        pltpu.async_copy(z2_hbm.at[src_v.at[j + NB + b]], bufs[b], gsems[b])

    return carry

  lax.fori_loop(0, KA // NB, body, 0)
  for b in range(NB):
    pltpu.make_async_copy(bufs[b], acc.at[dst_v.at[KA - NB + b]],
                          ssems[b]).wait()
  plsc.subcore_barrier()
  _striped_out_copy(s, acc, out_hbm.at[c])


_sc_agg = pl.kernel(
    _agg_body,
    out_type=jax.ShapeDtypeStruct((NC, N, HCH), jnp.float32),
    mesh=_mesh,
    scratch_types=[
        pltpu.VMEM((KA, C), jnp.int32),
        pltpu.VMEM((KA, C), jnp.int32),
        [pltpu.VMEM((C, HCH), jnp.float32) for _ in range(NB)],
        [pltpu.SemaphoreType.DMA for _ in range(NB)],
        [pltpu.SemaphoreType.DMA for _ in range(NB)],
        pltpu.VMEM_SHARED((NPAD, HCH), jnp.float32),
    ],
    compiler_params=pltpu.CompilerParams(use_tc_tiling_on_sc=False),
)


BN = 1000


def _scale_body(x_ref, d_ref, z_ref):
  deg = 1.0 + d_ref[0, :, 0:1] + d_ref[1, :, 0:1]
  z_ref[...] = x_ref[...] * lax.rsqrt(deg)


_tc_scale = pl.pallas_call(
    _scale_body,
    grid=(N // BN,),
    in_specs=[
        pl.BlockSpec((BN, CH), lambda i: (i, 0)),
        pl.BlockSpec((NC, BN, 16), lambda i: (0, i, 0)),
    ],
    out_specs=pl.BlockSpec((BN, CH), lambda i: (i, 0)),
    out_shape=jax.ShapeDtypeStruct((N, CH), jnp.float32),
)


def _final_body(a_ref, z_ref, d_ref, w_ref, b_ref, o_ref):
  deg = 1.0 + d_ref[0, :, 0:1] + d_ref[1, :, 0:1]
  agg = jnp.concatenate([a_ref[0], a_ref[1]], axis=1)
  t = (agg + z_ref[...]) * lax.rsqrt(deg)
  o_ref[...] = jnp.dot(t, w_ref[...],
                       preferred_element_type=jnp.float32) + b_ref[...]


_tc_final = pl.pallas_call(
    _final_body,
    grid=(N // BN,),
    in_specs=[
        pl.BlockSpec((NC, BN, HCH), lambda i: (0, i, 0)),
        pl.BlockSpec((BN, CH), lambda i: (i, 0)),
        pl.BlockSpec((NC, BN, 16), lambda i: (0, i, 0)),
        pl.BlockSpec((CH, CH), lambda i: (0, 0)),
        pl.BlockSpec((1, CH), lambda i: (0, 0)),
    ],
    out_specs=pl.BlockSpec((BN, CH), lambda i: (i, 0)),
    out_shape=jax.ShapeDtypeStruct((N, CH), jnp.float32),
)


@jax.jit
def kernel(x, edge_index, W, b):
  src = edge_index[0].astype(jnp.int32)
  dst = edge_index[1].astype(jnp.int32)

  epd = E // NW
  dstd = jnp.concatenate(
      [dst.reshape(NW, epd),
       jnp.full((NW, KD * C - epd), N, jnp.int32)], axis=1).reshape(NW, KD, C)

  epa = E // NS
  pada = KA * C - epa
  s2 = 2 * src.reshape(NS, epa)
  srca = jnp.stack([
      jnp.concatenate([s2, jnp.zeros((NS, pada), jnp.int32)], axis=1),
      jnp.concatenate([s2 + 1, jnp.ones((NS, pada), jnp.int32)], axis=1),
  ]).reshape(NC, NS, KA, C)
  dsta = jnp.concatenate(
      [dst.reshape(NS, epa),
       jnp.full((NS, pada), N, jnp.int32)], axis=1).reshape(NS, KA, C)

  zeros16 = jnp.zeros((NPAD, 16), jnp.float32)
  zeros64 = jnp.zeros((NPAD, HCH), jnp.float32)

  degp = _sc_deg(dstd, zeros16)
  z = _tc_scale(x, degp)
  accp = _sc_agg(z.reshape(2 * N, HCH), srca, dsta, zeros64)
  out = _tc_final(accp, z, degp, W, b.reshape(1, CH))
  return out


# speedup vs baseline: 21.7016x; 1.3842x over previous
import jax
import jax.numpy as jnp
from jax import lax
from jax.experimental import pallas as pl
from jax.experimental.pallas import tpu as pltpu
from jax.experimental.pallas import tpu_sc as plsc

N = 10000
E = 320000
CH = 128
HCH = CH // 2

NC = 2
NS = 16
NW = NC * NS

C = 128
KD = 80
KA = 160
NPH = 4
KQ = KA // NPH
NB = 4
NPAD = 10112
APAD = 10008
RPT = 632
OLAST = N - (NS - 1) * RPT
ALAST = APAD - (NS - 1) * RPT

_mesh = plsc.VectorSubcoreMesh(
    core_axis_name="c", subcore_axis_name="s", num_cores=NC, num_subcores=NS)


def _striped_copy(s, src_view, dst_view, last):
  @pl.when(s < NS - 1)
  def _():
    pltpu.sync_copy(src_view.at[pl.ds(s * RPT, RPT)],
                    dst_view.at[pl.ds(s * RPT, RPT)])

  @pl.when(s == NS - 1)
  def _():
    pltpu.sync_copy(src_view.at[pl.ds((NS - 1) * RPT, last)],
                    dst_view.at[pl.ds((NS - 1) * RPT, last)])


def _deg_body(dst_hbm, zeros_hbm, out_hbm, dst_v, ones_v, deg_sh, ssem):
  c = lax.axis_index("c")
  s = lax.axis_index("s")
  wid = s * NC + c
  pltpu.sync_copy(zeros_hbm.at[pl.ds(s * RPT, RPT)],
                  deg_sh.at[pl.ds(s * RPT, RPT)])
  pltpu.sync_copy(dst_hbm.at[wid], dst_v)

  def fill(i, carry):
    ones_v[i, :] = jnp.ones((16,), jnp.float32)
    return carry

  lax.fori_loop(0, C, fill, 0)
  plsc.subcore_barrier()

  def body(j, carry):
    pltpu.async_copy(ones_v, deg_sh.at[dst_v.at[j]], ssem, add=True)
    return carry

  lax.fori_loop(0, KD, body, 0)

  def drain(j, carry):
    pltpu.make_async_copy(ones_v, deg_sh.at[dst_v.at[j]], ssem).wait()
    return carry

  lax.fori_loop(0, KD, drain, 0)
  plsc.subcore_barrier()
  _striped_copy(s, deg_sh, out_hbm.at[c], OLAST)


_sc_deg = pl.kernel(
    _deg_body,
    out_type=jax.ShapeDtypeStruct((NC, N, 16), jnp.float32),
    mesh=_mesh,
    scratch_types=[
        pltpu.VMEM((KD, C), jnp.int32),
        pltpu.VMEM((C, 16), jnp.float32),
        pltpu.VMEM_SHARED((NPAD, 16), jnp.float32),
        pltpu.SemaphoreType.DMA,
    ],
)


def _agg_body(zflat_hbm, src_hbm, dst_hbm, zeros_hbm, out_hbm,
              src_v, dst_v, bufs, gsems, ssems, acc):
  c = lax.axis_index("c")
  s = lax.axis_index("s")
  _striped_copy(s, zeros_hbm, acc, ALAST)
  pltpu.sync_copy(src_hbm.at[c, s], src_v)
  pltpu.sync_copy(dst_hbm.at[s], dst_v)
  plsc.subcore_barrier()

  for b in range(NB):
    pltpu.async_copy(zflat_hbm.at[src_v.at[b]], bufs[b], gsems[b])

  def body(g, carry):
    j = g * NB
    for b in range(NB):
      pltpu.make_async_copy(zflat_hbm.at[src_v.at[j + b]], bufs[b],
                            gsems[b]).wait()
      pltpu.async_copy(bufs[b], acc.at[dst_v.at[j + b]], ssems[b],
                       add=True)

    @pl.when(g < KA // NB - 1)
    def _():
      for b in range(NB):
        pltpu.make_async_copy(bufs[b], acc.at[dst_v.at[j + b]],
                              ssems[b]).wait()
        pltpu.async_copy(zflat_hbm.at[src_v.at[j + NB + b]], bufs[b],
                         gsems[b])

    return carry

  lax.fori_loop(0, KA // NB, body, 0)
  for b in range(NB):
    pltpu.make_async_copy(bufs[b], acc.at[dst_v.at[KA - NB + b]],
                          ssems[b]).wait()

  plsc.subcore_barrier()
  _striped_copy(s, acc, out_hbm.at[c], OLAST)


_sc_agg = pl.kernel(
    _agg_body,
    out_type=jax.ShapeDtypeStruct((NC, N, HCH), jnp.float32),
    mesh=_mesh,
    scratch_types=[
        pltpu.VMEM((KA, C), jnp.int32),
        pltpu.VMEM((KA, C), jnp.int32),
        [pltpu.VMEM((C, HCH), jnp.float32) for _ in range(NB)],
        [pltpu.SemaphoreType.DMA for _ in range(NB)],
        [pltpu.SemaphoreType.DMA for _ in range(NB)],
        pltpu.VMEM_SHARED((APAD, HCH), jnp.float32),
    ],
    compiler_params=pltpu.CompilerParams(use_tc_tiling_on_sc=False),
)


BN = 1000


def _scale_body(x_ref, d_ref, z_ref):
  deg = 1.0 + d_ref[0, :, 0:1] + d_ref[1, :, 0:1]
  dinv = lax.rsqrt(deg)
  z_ref[0] = x_ref[:, :HCH] * dinv
  z_ref[1] = x_ref[:, HCH:] * dinv


_tc_scale = pl.pallas_call(
    _scale_body,
    grid=(N // BN,),
    in_specs=[
        pl.BlockSpec((BN, CH), lambda i: (i, 0)),
        pl.BlockSpec((NC, BN, 16), lambda i: (0, i, 0)),
    ],
    out_specs=pl.BlockSpec((NC, BN, HCH), lambda i: (0, i, 0)),
    out_shape=jax.ShapeDtypeStruct((NC, N, HCH), jnp.float32),
)


def _final_body(a_ref, z_ref, d_ref, w_ref, b_ref, o_ref):
  deg = 1.0 + d_ref[0, :, 0:1] + d_ref[1, :, 0:1]
  agg = jnp.concatenate([a_ref[0] + z_ref[0], a_ref[1] + z_ref[1]], axis=1)
  t = agg * lax.rsqrt(deg)
  o_ref[...] = jnp.dot(t, w_ref[...],
                       preferred_element_type=jnp.float32) + b_ref[...]


_tc_final = pl.pallas_call(
    _final_body,
    grid=(N // BN,),
    in_specs=[
        pl.BlockSpec((NC, BN, HCH), lambda i: (0, i, 0)),
        pl.BlockSpec((NC, BN, HCH), lambda i: (0, i, 0)),
        pl.BlockSpec((NC, BN, 16), lambda i: (0, i, 0)),
        pl.BlockSpec((CH, CH), lambda i: (0, 0)),
        pl.BlockSpec((1, CH), lambda i: (0, 0)),
    ],
    out_specs=pl.BlockSpec((BN, CH), lambda i: (i, 0)),
    out_shape=jax.ShapeDtypeStruct((N, CH), jnp.float32),
)


@jax.jit
def kernel(x, edge_index, W, b):
  src = edge_index[0].astype(jnp.int32)
  dst = edge_index[1].astype(jnp.int32)

  epd = E // NW
  dstd = jnp.concatenate(
      [dst.reshape(NW, epd),
       jnp.full((NW, KD * C - epd), N, jnp.int32)], axis=1).reshape(NW, KD, C)

  epa = E // NS
  pada = KA * C - epa
  srca = jnp.concatenate(
      [src.reshape(NS, epa),
       jnp.zeros((NS, pada), jnp.int32)], axis=1).reshape(NS, KA, C)
  srcb = jnp.stack([srca, srca + N])
  dsta = jnp.concatenate(
      [dst.reshape(NS, epa),
       jnp.full((NS, pada), N, jnp.int32)], axis=1).reshape(NS, KA, C)

  zeros16 = jnp.zeros((NPAD, 16), jnp.float32)
  zeros64 = jnp.zeros((APAD, HCH), jnp.float32)

  degp = _sc_deg(dstd, zeros16)
  zs = _tc_scale(x, degp)
  accp = _sc_agg(zs.reshape(NC * N, HCH), srcb, dsta, zeros64)
  out = _tc_final(accp, zs, degp, W, b.reshape(1, CH))
  return out


# speedup vs baseline: 22.2326x; 1.0245x over previous
import jax
import jax.numpy as jnp
from jax import lax
from jax.experimental import pallas as pl
from jax.experimental.pallas import tpu as pltpu
from jax.experimental.pallas import tpu_sc as plsc

N = 10000
E = 320000
CH = 128
HCH = CH // 2

NC = 2
NS = 16
NW = NC * NS

C = 128
KD = 80
KA = 160
NPH = 4
KQ = KA // NPH
NB = 5
NPAD = 10112
APAD = 10008
RPT = 632
OLAST = N - (NS - 1) * RPT
ALAST = APAD - (NS - 1) * RPT

_mesh = plsc.VectorSubcoreMesh(
    core_axis_name="c", subcore_axis_name="s", num_cores=NC, num_subcores=NS)


def _striped_copy(s, src_view, dst_view, last):
  @pl.when(s < NS - 1)
  def _():
    pltpu.sync_copy(src_view.at[pl.ds(s * RPT, RPT)],
                    dst_view.at[pl.ds(s * RPT, RPT)])

  @pl.when(s == NS - 1)
  def _():
    pltpu.sync_copy(src_view.at[pl.ds((NS - 1) * RPT, last)],
                    dst_view.at[pl.ds((NS - 1) * RPT, last)])


def _deg_body(dst_hbm, ones_hbm, zeros_hbm, out_hbm, dst_v, ones_v, deg_sh,
              ssem):
  c = lax.axis_index("c")
  s = lax.axis_index("s")
  wid = s * NC + c
  pltpu.sync_copy(zeros_hbm.at[pl.ds(s * RPT, RPT)],
                  deg_sh.at[pl.ds(s * RPT, RPT)])
  pltpu.sync_copy(dst_hbm.at[wid], dst_v)
  pltpu.sync_copy(ones_hbm, ones_v)
  plsc.subcore_barrier()

  def body(j, carry):
    pltpu.async_copy(ones_v, deg_sh.at[dst_v.at[j]], ssem, add=True)
    return carry

  lax.fori_loop(0, KD, body, 0)

  def drain(j, carry):
    pltpu.make_async_copy(ones_v, deg_sh.at[dst_v.at[j]], ssem).wait()
    return carry

  lax.fori_loop(0, KD, drain, 0)
  plsc.subcore_barrier()
  _striped_copy(s, deg_sh, out_hbm.at[c], OLAST)


_sc_deg = pl.kernel(
    _deg_body,
    out_type=jax.ShapeDtypeStruct((NC, N, 8), jnp.float32),
    mesh=_mesh,
    scratch_types=[
        pltpu.VMEM((KD, C), jnp.int32),
        pltpu.VMEM((C, 8), jnp.float32),
        pltpu.VMEM_SHARED((NPAD, 8), jnp.float32),
        pltpu.SemaphoreType.DMA,
    ],
    compiler_params=pltpu.CompilerParams(use_tc_tiling_on_sc=False),
)


def _agg_body(zflat_hbm, src_hbm, dst_hbm, zeros_hbm, out_hbm,
              src_v, dst_v, bufs, gsems, ssems, acc):
  c = lax.axis_index("c")
  s = lax.axis_index("s")
  _striped_copy(s, zeros_hbm, acc, ALAST)
  pltpu.sync_copy(src_hbm.at[c, s], src_v)
  pltpu.sync_copy(dst_hbm.at[s], dst_v)
  plsc.subcore_barrier()

  for b in range(NB):
    pltpu.async_copy(zflat_hbm.at[src_v.at[b]], bufs[b], gsems[b])

  def body(g, carry):
    j = g * NB
    for b in range(NB):
      pltpu.make_async_copy(zflat_hbm.at[src_v.at[j + b]], bufs[b],
                            gsems[b]).wait()
      pltpu.async_copy(bufs[b], acc.at[dst_v.at[j + b]], ssems[b],
                       add=True)

    @pl.when(g < KA // NB - 1)
    def _():
      for b in range(NB):
        pltpu.make_async_copy(bufs[b], acc.at[dst_v.at[j + b]],
                              ssems[b]).wait()
        pltpu.async_copy(zflat_hbm.at[src_v.at[j + NB + b]], bufs[b],
                         gsems[b])

    return carry

  lax.fori_loop(0, KA // NB, body, 0)
  for b in range(NB):
    pltpu.make_async_copy(bufs[b], acc.at[dst_v.at[KA - NB + b]],
                          ssems[b]).wait()

  plsc.subcore_barrier()
  _striped_copy(s, acc, out_hbm.at[c], OLAST)


_sc_agg = pl.kernel(
    _agg_body,
    out_type=jax.ShapeDtypeStruct((NC, N, HCH), jnp.float32),
    mesh=_mesh,
    scratch_types=[
        pltpu.VMEM((KA, C), jnp.int32),
        pltpu.VMEM((KA, C), jnp.int32),
        [pltpu.VMEM((C, HCH), jnp.float32) for _ in range(NB)],
        [pltpu.SemaphoreType.DMA for _ in range(NB)],
        [pltpu.SemaphoreType.DMA for _ in range(NB)],
        pltpu.VMEM_SHARED((APAD, HCH), jnp.float32),
    ],
    compiler_params=pltpu.CompilerParams(use_tc_tiling_on_sc=False),
)


BN = 1000


def _scale_body(x_ref, d_ref, z_ref):
  deg = 1.0 + d_ref[0, :, 0:1] + d_ref[1, :, 0:1]
  dinv = lax.rsqrt(deg)
  z_ref[0] = x_ref[:, :HCH] * dinv
  z_ref[1] = x_ref[:, HCH:] * dinv


_tc_scale = pl.pallas_call(
    _scale_body,
    grid=(N // BN,),
    in_specs=[
        pl.BlockSpec((BN, CH), lambda i: (i, 0)),
        pl.BlockSpec((NC, BN, 8), lambda i: (0, i, 0)),
    ],
    out_specs=pl.BlockSpec((NC, BN, HCH), lambda i: (0, i, 0)),
    out_shape=jax.ShapeDtypeStruct((NC, N, HCH), jnp.float32),
)


def _final_body(a_ref, z_ref, d_ref, w_ref, b_ref, o_ref):
  deg = 1.0 + d_ref[0, :, 0:1] + d_ref[1, :, 0:1]
  agg = jnp.concatenate([a_ref[0] + z_ref[0], a_ref[1] + z_ref[1]], axis=1)
  t = agg * lax.rsqrt(deg)
  o_ref[...] = jnp.dot(t, w_ref[...],
                       preferred_element_type=jnp.float32) + b_ref[...]


_tc_final = pl.pallas_call(
    _final_body,
    grid=(N // BN,),
    in_specs=[
        pl.BlockSpec((NC, BN, HCH), lambda i: (0, i, 0)),
        pl.BlockSpec((NC, BN, HCH), lambda i: (0, i, 0)),
        pl.BlockSpec((NC, BN, 8), lambda i: (0, i, 0)),
        pl.BlockSpec((CH, CH), lambda i: (0, 0)),
        pl.BlockSpec((1, CH), lambda i: (0, 0)),
    ],
    out_specs=pl.BlockSpec((BN, CH), lambda i: (i, 0)),
    out_shape=jax.ShapeDtypeStruct((N, CH), jnp.float32),
)


@jax.jit
def kernel(x, edge_index, W, b):
  src = edge_index[0].astype(jnp.int32)
  dst = edge_index[1].astype(jnp.int32)

  epd = E // NW
  dstd = jnp.concatenate(
      [dst.reshape(NW, epd),
       jnp.full((NW, KD * C - epd), N, jnp.int32)], axis=1).reshape(NW, KD, C)

  epa = E // NS
  pada = KA * C - epa
  srca = jnp.concatenate(
      [src.reshape(NS, epa),
       jnp.zeros((NS, pada), jnp.int32)], axis=1).reshape(NS, KA, C)
  srcb = jnp.stack([srca, srca + N])
  dsta = jnp.concatenate(
      [dst.reshape(NS, epa),
       jnp.full((NS, pada), N, jnp.int32)], axis=1).reshape(NS, KA, C)

  zeros8 = jnp.zeros((NPAD, 8), jnp.float32)
  ones8 = jnp.ones((C, 8), jnp.float32)
  zeros64 = jnp.zeros((APAD, HCH), jnp.float32)

  degp = _sc_deg(dstd, ones8, zeros8)
  zs = _tc_scale(x, degp)
  accp = _sc_agg(zs.reshape(NC * N, HCH), srcb, dsta, zeros64)
  out = _tc_final(accp, zs, degp, W, b.reshape(1, CH))
  return out
